# Initial kernel scaffold; baseline (speedup 1.0000x reference)
import jax
import jax.numpy as jnp
from jax.experimental import pallas as pl


def kernel(y, emb_weight):
    raise NotImplementedError("write your pallas kernel here")


# speedup vs baseline: 8.7954x; 8.7954x over previous
import functools

import jax
import jax.numpy as jnp
from jax import lax
from jax.experimental import pallas as pl
from jax.experimental.pallas import tpu as pltpu
from jax.experimental.pallas import tpu_sc as plsc

_B_ROWS = 4096
_SEQ = 200
_D = 128
_B = _B_ROWS * _SEQ
_NC = 2
_NS = 16
_NW = _NC * _NS
_BPW = _B // _NW
_CHUNK = 800
_NCHUNK = _BPW // _CHUNK


def _emb_body(table_hbm, idx_hbm, out_hbm, idx_v, rows_v, sem):
    wid = lax.axis_index("s") * _NC + lax.axis_index("c")
    base = wid * _BPW

    def body(i, carry):
        off = base + i * _CHUNK
        pltpu.sync_copy(idx_hbm.at[pl.ds(off, _CHUNK)], idx_v)
        pltpu.async_copy(table_hbm.at[idx_v], rows_v, sem).wait()
        pltpu.sync_copy(rows_v, out_hbm.at[pl.ds(off, _CHUNK)])
        return carry

    lax.fori_loop(0, _NCHUNK, body, 0)


@jax.jit
def kernel(y, emb_weight):
    yf = y.reshape(_B).astype(jnp.int32)
    mesh = plsc.VectorSubcoreMesh(core_axis_name="c", subcore_axis_name="s")
    k = pl.kernel(
        _emb_body,
        out_type=jax.ShapeDtypeStruct((_B, _D), jnp.float32),
        mesh=mesh,
        scratch_types=[
            pltpu.VMEM((_CHUNK,), jnp.int32),
            pltpu.VMEM((_CHUNK, _D), jnp.float32),
            pltpu.SemaphoreType.DMA,
        ],
    )
    out = k(emb_weight, yf)
    return out.reshape(_B_ROWS, _SEQ, _D)


# speedup vs baseline: 9.0659x; 1.0308x over previous
import jax
import jax.numpy as jnp
from jax import lax
from jax.experimental import pallas as pl
from jax.experimental.pallas import tpu as pltpu
from jax.experimental.pallas import tpu_sc as plsc

_B_ROWS = 4096
_SEQ = 200
_D = 128
_B = _B_ROWS * _SEQ
_NC = 2
_NS = 16
_NW = _NC * _NS
_BPW = _B // _NW
_NBUF = 4
_CHUNK = 200
_NCHUNK = _BPW // _CHUNK
_NG = _NCHUNK // _NBUF


def _emb_body(table_hbm, idx_hbm, out_hbm, idx_all,
              rows0, rows1, rows2, rows3, sg0, sg1, sg2, sg3,
              so0, so1, so2, so3):
    wid = lax.axis_index("s") * _NC + lax.axis_index("c")
    base = pl.multiple_of(wid * _BPW, _BPW)
    rows = (rows0, rows1, rows2, rows3)
    sg = (sg0, sg1, sg2, sg3)
    so = (so0, so1, so2, so3)

    pltpu.sync_copy(idx_hbm.at[pl.ds(base, _BPW)], idx_all)

    def gather_start(chunk, b):
        off = pl.multiple_of(chunk * _CHUNK, _CHUNK)
        pltpu.async_copy(
            table_hbm.at[idx_all.at[pl.ds(off, _CHUNK)]], rows[b], sg[b])

    def gather_wait(b):
        pltpu.make_async_copy(
            table_hbm.at[idx_all.at[pl.ds(0, _CHUNK)]], rows[b], sg[b]).wait()

    def scatter_start(chunk, b):
        off = pl.multiple_of(base + chunk * _CHUNK, _CHUNK)
        return pltpu.async_copy(rows[b], out_hbm.at[pl.ds(off, _CHUNK)], so[b])

    for b in range(_NBUF):
        gather_start(b, b)

    def body(g, carry):
        outs = []
        for b in range(_NBUF):
            gather_wait(b)
            outs.append(scatter_start(g * _NBUF + b, b))
        for b in range(_NBUF):
            outs[b].wait()
            gather_start((g + 1) * _NBUF + b, b)
        return carry

    lax.fori_loop(0, _NG - 1, body, 0)

    outs = []
    for b in range(_NBUF):
        gather_wait(b)
        outs.append(scatter_start((_NG - 1) * _NBUF + b, b))
    for o in outs:
        o.wait()


@jax.jit
def kernel(y, emb_weight):
    yf = y.reshape(_B).astype(jnp.int32)
    mesh = plsc.VectorSubcoreMesh(core_axis_name="c", subcore_axis_name="s")
    k = pl.kernel(
        _emb_body,
        out_type=jax.ShapeDtypeStruct((_B, _D), jnp.float32),
        mesh=mesh,
        scratch_types=(
            [pltpu.VMEM((_BPW,), jnp.int32)]
            + [pltpu.VMEM((_CHUNK, _D), jnp.float32)] * _NBUF
            + [pltpu.SemaphoreType.DMA] * (2 * _NBUF)
        ),
    )
    out = k(emb_weight, yf)
    return out.reshape(_B_ROWS, _SEQ, _D)
